# baseline (speedup 1.0000x reference)
import functools

import jax
import jax.numpy as jnp
from jax import lax
from jax.experimental import pallas as pl
from jax.experimental.pallas import tpu as pltpu
from jax.experimental.pallas import tpu_sc as plsc

NUM_EXPERTS = 8
HIDDEN = 1024

_NC = 2
_NS = 16
_L = 16
_NW = _NC * _NS

_BLK = 2048
_EBLK = 4096


def _logits_body(x_ref, w_ref, out_ref):
    out_ref[...] = lax.dot_general(
        w_ref[...], x_ref[...],
        dimension_numbers=(((1,), (1,)), ((), ())),
        preferred_element_type=jnp.float32,
    )


def _tc_logits(x, w, n_chunk, tok0):
    grid = n_chunk // _BLK
    blk0 = tok0 // _BLK
    return pl.pallas_call(
        _logits_body,
        grid=(grid,),
        in_specs=[
            pl.BlockSpec((_BLK, HIDDEN), lambda i: (blk0 + i, 0)),
            pl.BlockSpec((NUM_EXPERTS, HIDDEN), lambda i: (0, 0)),
        ],
        out_specs=pl.BlockSpec((NUM_EXPERTS, _BLK), lambda i: (0, i)),
        out_shape=jax.ShapeDtypeStruct((NUM_EXPERTS, n_chunk), jnp.float32),
        compiler_params=pltpu.CompilerParams(
            dimension_semantics=("arbitrary",),
        ),
    )(x, w)


def _stable_sigmoid(x):
    e = jnp.exp(-jnp.abs(x))
    return jnp.where(x >= 0, 1.0 / (1.0 + e), e / (1.0 + e))


def _make_router(n_tokens):
    t_per_w = n_tokens // _NW
    groups = t_per_w // _L
    mesh = plsc.VectorSubcoreMesh(
        core_axis_name="c", subcore_axis_name="s",
        num_cores=_NC, num_subcores=_NS,
    )

    @functools.partial(
        pl.kernel,
        out_type=jax.ShapeDtypeStruct((NUM_EXPERTS, n_tokens), jnp.float32),
        mesh=mesh,
        compiler_params=pltpu.CompilerParams(
            needs_layout_passes=False, use_tc_tiling_on_sc=False,
        ),
        scratch_types=[
            pltpu.VMEM((NUM_EXPERTS * t_per_w,), jnp.float32),
            pltpu.VMEM((4 * t_per_w,), jnp.float32),
        ],
    )
    def router(logits_hbm, img_hbm, l_v, img_v):
        wid = lax.axis_index("s") * _NC + lax.axis_index("c")
        base = wid * t_per_w
        for e in range(NUM_EXPERTS):
            pltpu.sync_copy(
                logits_hbm.at[e, pl.ds(base, t_per_w)],
                l_v.at[pl.ds(e * t_per_w, t_per_w)],
            )

        neg_inf = jnp.float32(float("-inf"))
        zeros = jnp.zeros((_L,), jnp.int32)

        def body(g, carry):
            o = g * _L
            lv = [
                l_v[pl.ds(e * t_per_w + o, _L)]
                for e in range(NUM_EXPERTS)
            ]
            m1 = lv[0]
            for e in range(1, NUM_EXPERTS):
                m1 = jnp.maximum(m1, lv[e])
            i1 = zeros
            for e in reversed(range(NUM_EXPERTS)):
                i1 = jnp.where(lv[e] == m1, e, i1)
            lm = [
                jnp.where(i1 == e, neg_inf, lv[e])
                for e in range(NUM_EXPERTS)
            ]
            m2 = lm[0]
            for e in range(1, NUM_EXPERTS):
                m2 = jnp.maximum(m2, lm[e])
            i2 = zeros
            for e in reversed(range(NUM_EXPERTS)):
                i2 = jnp.where(lm[e] == m2, e, i2)
            s1 = _stable_sigmoid(m1)
            s2 = _stable_sigmoid(m2)
            denom = s1 + s2
            img_v[pl.ds(o, _L)] = s1 / denom
            img_v[pl.ds(t_per_w + o, _L)] = s2 / denom
            img_v[pl.ds(2 * t_per_w + o, _L)] = i1.astype(jnp.float32)
            img_v[pl.ds(3 * t_per_w + o, _L)] = i2.astype(jnp.float32)
            return carry

        lax.fori_loop(0, groups, body, 0)
        for r in range(4):
            pltpu.sync_copy(
                img_v.at[pl.ds(r * t_per_w, t_per_w)],
                img_hbm.at[r, pl.ds(base, t_per_w)],
            )

    return router


def _epilogue_body(img_ref, w_ref, i_ref):
    t = img_ref[...].T
    w_ref[...] = t[:, :2]
    i_ref[...] = t[:, 2:4].astype(jnp.int32)


def _tc_epilogue(img, n_tokens):
    grid = n_tokens // _EBLK
    return pl.pallas_call(
        _epilogue_body,
        grid=(grid,),
        in_specs=[
            pl.BlockSpec((NUM_EXPERTS, _EBLK), lambda i: (0, i)),
        ],
        out_specs=[
            pl.BlockSpec((_EBLK, 2), lambda i: (i, 0)),
            pl.BlockSpec((_EBLK, 2), lambda i: (i, 0)),
        ],
        out_shape=[
            jax.ShapeDtypeStruct((n_tokens, 2), jnp.float32),
            jax.ShapeDtypeStruct((n_tokens, 2), jnp.int32),
        ],
    )(img)


def kernel(hidden_states, gate_weight):
    b, s, d = hidden_states.shape
    n_tokens = b * s
    x = hidden_states.reshape(n_tokens, d)
    x = x.astype(jnp.float32)
    w = gate_weight.astype(jnp.float32)
    n_chunks = 4
    chunk = n_tokens // n_chunks
    router = _make_router(chunk)
    imgs = []
    for k in range(n_chunks):
        logits_k = _tc_logits(x, w, chunk, k * chunk)
        imgs.append(router(logits_k))
    img = jnp.concatenate(imgs, axis=1)
    routing_weights = img[:2, :].T
    indices = img[2:4, :].T.astype(jnp.int32)
    return routing_weights, indices


# speedup vs baseline: 1.1882x; 1.1882x over previous
import functools

import jax
import jax.numpy as jnp
from jax import lax
from jax.experimental import pallas as pl
from jax.experimental.pallas import tpu as pltpu
from jax.experimental.pallas import tpu_sc as plsc

NUM_EXPERTS = 8
HIDDEN = 1024

_NC = 2
_NS = 16
_L = 16
_NW = _NC * _NS

_BLK = 4096
_EBLK = 4096


def _logits_body(x_ref, w_ref, out_ref):
    out_ref[...] = lax.dot_general(
        w_ref[...], x_ref[...],
        dimension_numbers=(((1,), (1,)), ((), ())),
        preferred_element_type=jnp.float32,
    )


def _tc_logits(x, w, n_chunk, tok0):
    grid = n_chunk // _BLK
    blk0 = tok0 // _BLK
    return pl.pallas_call(
        _logits_body,
        grid=(grid,),
        in_specs=[
            pl.BlockSpec((_BLK, HIDDEN), lambda i: (blk0 + i, 0)),
            pl.BlockSpec((NUM_EXPERTS, HIDDEN), lambda i: (0, 0)),
        ],
        out_specs=pl.BlockSpec((NUM_EXPERTS, _BLK), lambda i: (0, i)),
        out_shape=jax.ShapeDtypeStruct((NUM_EXPERTS, n_chunk), jnp.float32),
        compiler_params=pltpu.CompilerParams(
            dimension_semantics=("arbitrary",),
        ),
    )(x, w)


def _stable_sigmoid(x):
    e = jnp.exp(-jnp.abs(x))
    return jnp.where(x >= 0, 1.0 / (1.0 + e), e / (1.0 + e))


def _make_router(n_tokens):
    t_per_w = n_tokens // _NW
    groups = t_per_w // _L
    mesh = plsc.VectorSubcoreMesh(
        core_axis_name="c", subcore_axis_name="s",
        num_cores=_NC, num_subcores=_NS,
    )
    unroll = 4
    assert groups % unroll == 0

    @functools.partial(
        pl.kernel,
        out_type=jax.ShapeDtypeStruct((NUM_EXPERTS, n_tokens), jnp.float32),
        mesh=mesh,
        compiler_params=pltpu.CompilerParams(
            needs_layout_passes=False, use_tc_tiling_on_sc=False,
        ),
        scratch_types=[
            pltpu.VMEM((NUM_EXPERTS * t_per_w,), jnp.float32),
            pltpu.VMEM((4 * t_per_w,), jnp.float32),
            pltpu.SemaphoreType.DMA,
        ],
    )
    def router(logits_hbm, img_hbm, l_v, img_v, sem):
        wid = lax.axis_index("s") * _NC + lax.axis_index("c")
        base = wid * t_per_w
        copies = [
            pltpu.async_copy(
                logits_hbm.at[e, pl.ds(base, t_per_w)],
                l_v.at[pl.ds(e * t_per_w, t_per_w)],
                sem,
            )
            for e in range(NUM_EXPERTS)
        ]
        for c in copies:
            c.wait()

        neg_inf = jnp.float32(float("-inf"))
        zeros = jnp.zeros((_L,), jnp.int32)

        def one_group(o):
            lv = [
                l_v[pl.ds(e * t_per_w + o, _L)]
                for e in range(NUM_EXPERTS)
            ]
            m1 = lv[0]
            for e in range(1, NUM_EXPERTS):
                m1 = jnp.maximum(m1, lv[e])
            i1 = zeros
            for e in reversed(range(NUM_EXPERTS)):
                i1 = jnp.where(lv[e] == m1, e, i1)
            lm = [
                jnp.where(i1 == e, neg_inf, lv[e])
                for e in range(NUM_EXPERTS)
            ]
            m2 = lm[0]
            for e in range(1, NUM_EXPERTS):
                m2 = jnp.maximum(m2, lm[e])
            i2 = zeros
            for e in reversed(range(NUM_EXPERTS)):
                i2 = jnp.where(lm[e] == m2, e, i2)
            s1 = _stable_sigmoid(m1)
            s2 = _stable_sigmoid(m2)
            denom = s1 + s2
            img_v[pl.ds(o, _L)] = s1 / denom
            img_v[pl.ds(t_per_w + o, _L)] = s2 / denom
            img_v[pl.ds(2 * t_per_w + o, _L)] = i1.astype(jnp.float32)
            img_v[pl.ds(3 * t_per_w + o, _L)] = i2.astype(jnp.float32)

        def body(g, carry):
            o = g * (_L * unroll)
            for u in range(unroll):
                one_group(o + u * _L)
            return carry

        lax.fori_loop(0, groups // unroll, body, 0)
        out_copies = [
            pltpu.async_copy(
                img_v.at[pl.ds(r * t_per_w, t_per_w)],
                img_hbm.at[r, pl.ds(base, t_per_w)],
                sem,
            )
            for r in range(4)
        ]
        for c in out_copies:
            c.wait()

    return router


def _epilogue_body(img_ref, w_ref, i_ref):
    t = img_ref[...].T
    w_ref[...] = t[:, :2]
    i_ref[...] = t[:, 2:4].astype(jnp.int32)


def _tc_epilogue(img, n_tokens):
    grid = n_tokens // _EBLK
    return pl.pallas_call(
        _epilogue_body,
        grid=(grid,),
        in_specs=[
            pl.BlockSpec((NUM_EXPERTS, _EBLK), lambda i: (0, i)),
        ],
        out_specs=[
            pl.BlockSpec((_EBLK, 2), lambda i: (i, 0)),
            pl.BlockSpec((_EBLK, 2), lambda i: (i, 0)),
        ],
        out_shape=[
            jax.ShapeDtypeStruct((n_tokens, 2), jnp.float32),
            jax.ShapeDtypeStruct((n_tokens, 2), jnp.int32),
        ],
    )(img)


def kernel(hidden_states, gate_weight):
    b, s, d = hidden_states.shape
    n_tokens = b * s
    x = hidden_states.reshape(n_tokens, d)
    x = x.astype(jnp.float32)
    w = gate_weight.astype(jnp.float32)
    logits = _tc_logits(x, w, n_tokens, 0)
    img = _make_router(n_tokens)(logits)
    routing_weights = img[:2, :].T
    indices = img[2:4, :].T.astype(jnp.int32)
    return routing_weights, indices


# speedup vs baseline: 1.1949x; 1.0056x over previous
import functools

import jax
import jax.numpy as jnp
from jax import lax
from jax.experimental import pallas as pl
from jax.experimental.pallas import tpu as pltpu
from jax.experimental.pallas import tpu_sc as plsc

NUM_EXPERTS = 8
HIDDEN = 1024

_NC = 2
_NS = 16
_L = 16
_NW = _NC * _NS

_BLK = 4096
_EBLK = 4096


def _logits_body(x_ref, w_ref, out_ref):
    lg = lax.dot_general(
        w_ref[...], x_ref[...],
        dimension_numbers=(((1,), (1,)), ((), ())),
        preferred_element_type=jnp.float32,
    )
    out_ref[...] = lg.reshape(NUM_EXPERTS, _BLK // 128, 128).transpose(1, 0, 2)


def _tc_logits(x, w, n_chunk, tok0):
    grid = n_chunk // _BLK
    blk0 = tok0 // _BLK
    return pl.pallas_call(
        _logits_body,
        grid=(grid,),
        in_specs=[
            pl.BlockSpec((_BLK, HIDDEN), lambda i: (blk0 + i, 0)),
            pl.BlockSpec((NUM_EXPERTS, HIDDEN), lambda i: (0, 0)),
        ],
        out_specs=pl.BlockSpec(
            (_BLK // 128, NUM_EXPERTS, 128), lambda i: (i, 0, 0)
        ),
        out_shape=jax.ShapeDtypeStruct(
            (n_chunk // 128, NUM_EXPERTS, 128), jnp.float32
        ),
        compiler_params=pltpu.CompilerParams(
            dimension_semantics=("arbitrary",),
        ),
    )(x, w)


def _stable_sigmoid(x):
    e = jnp.exp(-jnp.abs(x))
    return jnp.where(x >= 0, 1.0 / (1.0 + e), e / (1.0 + e))


def _make_router(n_tokens):
    t_per_w = n_tokens // _NW
    groups = t_per_w // _L
    mesh = plsc.VectorSubcoreMesh(
        core_axis_name="c", subcore_axis_name="s",
        num_cores=_NC, num_subcores=_NS,
    )
    unroll = 4
    assert groups % unroll == 0

    @functools.partial(
        pl.kernel,
        out_type=jax.ShapeDtypeStruct((NUM_EXPERTS, n_tokens), jnp.float32),
        mesh=mesh,
        compiler_params=pltpu.CompilerParams(
            needs_layout_passes=False, use_tc_tiling_on_sc=False,
        ),
        scratch_types=[
            pltpu.VMEM((t_per_w // 128, NUM_EXPERTS, 128), jnp.float32),
            pltpu.VMEM((4 * t_per_w,), jnp.float32),
            pltpu.SemaphoreType.DMA,
        ],
    )
    def router(logits_hbm, img_hbm, l_v, img_v, sem):
        wid = lax.axis_index("s") * _NC + lax.axis_index("c")
        base = wid * t_per_w
        pltpu.sync_copy(
            logits_hbm.at[pl.ds(base // 128, t_per_w // 128)], l_v
        )

        neg_inf = jnp.float32(float("-inf"))
        zeros = jnp.zeros((_L,), jnp.int32)

        def one_group(j, sub):
            o = sub * _L
            lv = [
                l_v[j, e, pl.ds(o, _L)]
                for e in range(NUM_EXPERTS)
            ]
            m1 = lv[0]
            for e in range(1, NUM_EXPERTS):
                m1 = jnp.maximum(m1, lv[e])
            i1 = zeros
            for e in reversed(range(NUM_EXPERTS)):
                i1 = jnp.where(lv[e] == m1, e, i1)
            lm = [
                jnp.where(i1 == e, neg_inf, lv[e])
                for e in range(NUM_EXPERTS)
            ]
            m2 = lm[0]
            for e in range(1, NUM_EXPERTS):
                m2 = jnp.maximum(m2, lm[e])
            i2 = zeros
            for e in reversed(range(NUM_EXPERTS)):
                i2 = jnp.where(lm[e] == m2, e, i2)
            s1 = _stable_sigmoid(m1)
            s2 = _stable_sigmoid(m2)
            denom = s1 + s2
            go = j * 128 + o
            img_v[pl.ds(go, _L)] = s1 / denom
            img_v[pl.ds(t_per_w + go, _L)] = s2 / denom
            img_v[pl.ds(2 * t_per_w + go, _L)] = i1.astype(jnp.float32)
            img_v[pl.ds(3 * t_per_w + go, _L)] = i2.astype(jnp.float32)

        def body(j, carry):
            for sub in range(8):
                one_group(j, sub)
            return carry

        lax.fori_loop(0, t_per_w // 128, body, 0)
        out_copies = [
            pltpu.async_copy(
                img_v.at[pl.ds(r * t_per_w, t_per_w)],
                img_hbm.at[r, pl.ds(base, t_per_w)],
                sem,
            )
            for r in range(4)
        ]
        for c in out_copies:
            c.wait()

    return router


def _epilogue_body(img_ref, w_ref, i_ref):
    t = img_ref[...].T
    w_ref[...] = t[:, :2]
    i_ref[...] = t[:, 2:4].astype(jnp.int32)


def _tc_epilogue(img, n_tokens):
    grid = n_tokens // _EBLK
    return pl.pallas_call(
        _epilogue_body,
        grid=(grid,),
        in_specs=[
            pl.BlockSpec((NUM_EXPERTS, _EBLK), lambda i: (0, i)),
        ],
        out_specs=[
            pl.BlockSpec((_EBLK, 2), lambda i: (i, 0)),
            pl.BlockSpec((_EBLK, 2), lambda i: (i, 0)),
        ],
        out_shape=[
            jax.ShapeDtypeStruct((n_tokens, 2), jnp.float32),
            jax.ShapeDtypeStruct((n_tokens, 2), jnp.int32),
        ],
    )(img)


def kernel(hidden_states, gate_weight):
    b, s, d = hidden_states.shape
    n_tokens = b * s
    x = hidden_states.reshape(n_tokens, d)
    x = x.astype(jnp.float32)
    w = gate_weight.astype(jnp.float32)
    logits = _tc_logits(x, w, n_tokens, 0)
    img = _make_router(n_tokens)(logits)
    routing_weights = img[:2, :].T
    indices = img[2:4, :].T.astype(jnp.int32)
    return routing_weights, indices
